# Initial kernel scaffold; baseline (speedup 1.0000x reference)
import jax
import jax.numpy as jnp
from jax.experimental import pallas as pl


def kernel(x, W):
    raise NotImplementedError("write your pallas kernel here")


# speedup vs baseline: 2.0601x; 2.0601x over previous
import functools
import numpy as np
import jax
import jax.numpy as jnp
from jax import lax
from jax.experimental import pallas as pl
from jax.experimental.pallas import tpu as pltpu
from jax.experimental.pallas import tpu_sc as plsc

_VOCAB = 100000
_DEPTH = 64
_BATCH = 4096
_SEQ = 200


def _positional_table():
    effective_depth = _DEPTH / 2
    depth_vector = np.repeat(np.arange(effective_depth), 2)
    frequency_vector = 1 / 10000 ** (2 * depth_vector / _DEPTH)
    sequence_vector = np.arange(_SEQ)
    pos = sequence_vector.reshape([-1, 1]) * frequency_vector.reshape([1, -1])
    pos[:, ::2] = np.sin(pos[:, ::2])
    pos[:, 1::2] = np.cos(pos[:, 1::2])
    pos = pos.astype(np.float32)
    return np.concatenate([pos, pos], axis=0)


_NC = 2
_NS = 16
_NW = _NC * _NS

_N = _BATCH * _SEQ
_PER_W = _N // _NW
_CHUNK = 128
_NCHUNK = _PER_W // _CHUNK


def _sc_body(w_hbm, idx_hbm, pos_hbm, out_hbm, idx_v, rows_v, pos_v, sem):
    wid = lax.axis_index("s") * _NC + lax.axis_index("c")
    base = wid * _PER_W

    pltpu.sync_copy(pos_hbm, pos_v)

    scale = jnp.float32(np.sqrt(float(_DEPTH)))

    def chunk_body(c, _):
        off = base + c * _CHUNK
        pltpu.sync_copy(idx_hbm.at[pl.ds(off, _CHUNK)], idx_v)
        pltpu.async_copy(w_hbm.at[idx_v], rows_v, sem).wait()
        r = lax.rem(c * _CHUNK, _SEQ)

        def elem_body(e, _):
            pe = r + e
            for p in range(_DEPTH // 16):
                sl = pl.ds(p * 16, 16)
                rows_v[e, sl] = rows_v[e, sl] * scale + pos_v[pe, sl]
            return 0

        lax.fori_loop(0, _CHUNK, elem_body, 0)
        pltpu.sync_copy(rows_v, out_hbm.at[pl.ds(off, _CHUNK)])
        return 0

    lax.fori_loop(0, _NCHUNK, chunk_body, 0)


@jax.jit
def _embed(x, W):
    pos2 = jnp.asarray(_positional_table())
    idx = x.reshape(-1)
    mesh = plsc.VectorSubcoreMesh(core_axis_name="c", subcore_axis_name="s")
    out = pl.kernel(
        _sc_body,
        mesh=mesh,
        compiler_params=pltpu.CompilerParams(use_tc_tiling_on_sc=False),
        out_type=jax.ShapeDtypeStruct((_N, _DEPTH), jnp.float32),
        scratch_types=[
            pltpu.VMEM((_CHUNK,), jnp.int32),
            pltpu.VMEM((_CHUNK, _DEPTH), jnp.float32),
            pltpu.VMEM((2 * _SEQ, _DEPTH), jnp.float32),
            pltpu.SemaphoreType.DMA,
        ],
    )(W, idx, pos2)
    return out.reshape(_BATCH, _SEQ, _DEPTH)


def kernel(x, W):
    return _embed(x, W)


# speedup vs baseline: 4.2094x; 2.0433x over previous
import numpy as np
import jax
import jax.numpy as jnp
from jax import lax
from jax.experimental import pallas as pl
from jax.experimental.pallas import tpu as pltpu
from jax.experimental.pallas import tpu_sc as plsc

_VOCAB = 100000
_DEPTH = 64
_BATCH = 4096
_SEQ = 200


def _positional_table():
    effective_depth = _DEPTH / 2
    depth_vector = np.repeat(np.arange(effective_depth), 2)
    frequency_vector = 1 / 10000 ** (2 * depth_vector / _DEPTH)
    sequence_vector = np.arange(_SEQ)
    pos = sequence_vector.reshape([-1, 1]) * frequency_vector.reshape([1, -1])
    pos[:, ::2] = np.sin(pos[:, ::2])
    pos[:, 1::2] = np.cos(pos[:, 1::2])
    pos = pos.astype(np.float32)
    return np.concatenate([pos, pos], axis=0)


_NC = 2
_NS = 16
_NW = _NC * _NS

_N = _BATCH * _SEQ
_PER_W = _N // _NW
_CHUNK = 128
_NCHUNK = _PER_W // _CHUNK
_NBUF = 4
_LOOK = 2


def _sc_body(w_hbm, idx_hbm, pos_hbm, out_hbm, idx_v, pos_v, rows_v,
             gsem, ssem):
    wid = lax.axis_index("s") * _NC + lax.axis_index("c")
    cbase = wid * _NCHUNK
    ebase = wid * _PER_W

    pltpu.sync_copy(pos_hbm, pos_v)
    pltpu.sync_copy(idx_hbm.at[pl.ds(cbase, _NCHUNK)], idx_v)

    scale = jnp.float32(np.sqrt(float(_DEPTH)))

    def start_gather(c, s):
        pltpu.async_copy(w_hbm.at[idx_v.at[c]], rows_v.at[s], gsem[s])

    def wait_gather(c, s):
        pltpu.make_async_copy(w_hbm.at[idx_v.at[c]], rows_v.at[s],
                              gsem[s]).wait()

    def out_slice(c):
        return out_hbm.at[pl.ds(ebase + c * _CHUNK, _CHUNK)]

    def start_scatter(c, s):
        pltpu.async_copy(rows_v.at[s], out_slice(c), ssem[s])

    def wait_scatter(c, s):
        pltpu.make_async_copy(rows_v.at[s], out_slice(c), ssem[s]).wait()

    for s in range(_LOOK):
        start_gather(s, s)

    def outer(o, _):
        for b in range(_NBUF):
            g = o * _NBUF + b
            sp = (b + _LOOK) % _NBUF

            @pl.when(g + _LOOK < _NCHUNK)
            def _():
                @pl.when(g >= _NBUF - _LOOK)
                def _():
                    wait_scatter(g + _LOOK - _NBUF, sp)
                start_gather(g + _LOOK, sp)

            wait_gather(g, b)

            r = lax.rem(g * _CHUNK, _SEQ)

            @plsc.parallel_loop(0, _CHUNK, unroll=4)
            def _(e):
                pe = r + e
                for p in range(_DEPTH // 16):
                    sl = pl.ds(p * 16, 16)
                    rows_v[b, e, sl] = rows_v[b, e, sl] * scale + pos_v[pe, sl]

            start_scatter(g, b)
        return 0

    lax.fori_loop(0, _NCHUNK // _NBUF, outer, 0)

    for s in range(_NBUF):
        wait_scatter(_NCHUNK - _NBUF + s, s)


@jax.jit
def _embed(x, W):
    pos2 = jnp.asarray(_positional_table())
    idx = x.reshape(_N // _CHUNK, _CHUNK)
    mesh = plsc.VectorSubcoreMesh(core_axis_name="c", subcore_axis_name="s")
    out = pl.kernel(
        _sc_body,
        mesh=mesh,
        compiler_params=pltpu.CompilerParams(use_tc_tiling_on_sc=False),
        out_type=jax.ShapeDtypeStruct((_N, _DEPTH), jnp.float32),
        scratch_types=[
            pltpu.VMEM((_NCHUNK, _CHUNK), jnp.int32),
            pltpu.VMEM((2 * _SEQ, _DEPTH), jnp.float32),
            pltpu.VMEM((_NBUF, _CHUNK, _DEPTH), jnp.float32),
            [pltpu.SemaphoreType.DMA] * _NBUF,
            [pltpu.SemaphoreType.DMA] * _NBUF,
        ],
    )(W, idx, pos2)
    return out.reshape(_BATCH, _SEQ, _DEPTH)


def kernel(x, W):
    return _embed(x, W)


# speedup vs baseline: 4.2477x; 1.0091x over previous
import numpy as np
import jax
import jax.numpy as jnp
from jax import lax
from jax.experimental import pallas as pl
from jax.experimental.pallas import tpu as pltpu
from jax.experimental.pallas import tpu_sc as plsc

_VOCAB = 100000
_DEPTH = 64
_BATCH = 4096
_SEQ = 200


def _positional_table():
    effective_depth = _DEPTH / 2
    depth_vector = np.repeat(np.arange(effective_depth), 2)
    frequency_vector = 1 / 10000 ** (2 * depth_vector / _DEPTH)
    sequence_vector = np.arange(_SEQ)
    pos = sequence_vector.reshape([-1, 1]) * frequency_vector.reshape([1, -1])
    pos[:, ::2] = np.sin(pos[:, ::2])
    pos[:, 1::2] = np.cos(pos[:, 1::2])
    return pos.astype(np.float32)


_NC = 2
_NS = 16
_NW = _NC * _NS

_SPW = _BATCH // _NW
_SPLITS = ((0, 128), (128, 72))
_NBUF = 4
_LOOK = 2


def _sc_body(w_hbm, x_hbm, pos_hbm, out_hbm, idx_v, pos_v, rows_v,
             gsem, ssem):
    wid = lax.axis_index("s") * _NC + lax.axis_index("c")
    sbase = wid * _SPW

    pltpu.sync_copy(pos_hbm, pos_v)
    pltpu.sync_copy(x_hbm.at[pl.ds(sbase, _SPW)], idx_v)

    scale = jnp.float32(np.sqrt(float(_DEPTH)))

    def gather_part(t, s, off, n):
        return pltpu.make_async_copy(
            w_hbm.at[idx_v.at[t, pl.ds(off, n)]],
            rows_v.at[s, pl.ds(off, n)],
            gsem[s])

    def start_gather(t, s):
        for off, n in _SPLITS:
            gather_part(t, s, off, n).start()

    def wait_gather(t, s):
        for off, n in _SPLITS:
            gather_part(t, s, off, n).wait()

    def scatter(t, s):
        return pltpu.make_async_copy(rows_v.at[s], out_hbm.at[sbase + t],
                                     ssem[s])

    for s in range(_LOOK):
        start_gather(s, s)

    def outer(o, _):
        for b in range(_NBUF):
            t = o * _NBUF + b
            sp = (b + _LOOK) % _NBUF

            @pl.when(t + _LOOK < _SPW)
            def _():
                @pl.when(t >= _NBUF - _LOOK)
                def _():
                    scatter(t + _LOOK - _NBUF, sp).wait()
                start_gather(t + _LOOK, sp)

            wait_gather(t, b)

            @plsc.parallel_loop(0, _SEQ, unroll=4)
            def _(e):
                for p in range(_DEPTH // 16):
                    sl = pl.ds(p * 16, 16)
                    rows_v[b, e, sl] = rows_v[b, e, sl] * scale + pos_v[e, sl]

            scatter(t, b).start()
        return 0

    lax.fori_loop(0, _SPW // _NBUF, outer, 0)

    for s in range(_NBUF):
        scatter(_SPW - _NBUF + s, s).wait()


@jax.jit
def _embed(x, W):
    pos = jnp.asarray(_positional_table())
    mesh = plsc.VectorSubcoreMesh(core_axis_name="c", subcore_axis_name="s")
    out = pl.kernel(
        _sc_body,
        mesh=mesh,
        compiler_params=pltpu.CompilerParams(use_tc_tiling_on_sc=False),
        out_type=jax.ShapeDtypeStruct((_BATCH, _SEQ, _DEPTH), jnp.float32),
        scratch_types=[
            pltpu.VMEM((_SPW, _SEQ), jnp.int32),
            pltpu.VMEM((_SEQ, _DEPTH), jnp.float32),
            pltpu.VMEM((_NBUF, _SEQ, _DEPTH), jnp.float32),
            [pltpu.SemaphoreType.DMA] * _NBUF,
            [pltpu.SemaphoreType.DMA] * _NBUF,
        ],
    )(W, x, pos)
    return out


def kernel(x, W):
    return _embed(x, W)
